# baseline (speedup 1.0000x reference)
import functools

import jax
import jax.numpy as jnp
from jax import lax
from jax.experimental import pallas as pl
from jax.experimental.pallas import tpu as pltpu
from jax.experimental.pallas import tpu_sc as plsc

EMB = 64
NC = 2
NS = 16
NW = NC * NS
BL = 128


def _relu_t_body(w_ref, o_ref):
    o_ref[...] = jnp.maximum(w_ref[...], 0.0).T


def _relu_t_table(w):
    return pl.pallas_call(
        _relu_t_body,
        out_shape=jax.ShapeDtypeStruct((w.shape[1], w.shape[0]), w.dtype),
    )(w)


def _make_gather(nb, nh, vocab):
    nbt = nb // BL
    assert nbt == NW and nh % 2 == 0
    mesh = plsc.VectorSubcoreMesh(core_axis_name="c", subcore_axis_name="s")

    @functools.partial(
        pl.kernel,
        mesh=mesh,
        compiler_params=pltpu.CompilerParams(
            use_tc_tiling_on_sc=False, needs_layout_passes=False),
        out_type=jax.ShapeDtypeStruct((nh, 8, nbt, 8, BL), jnp.float32),
        scratch_types=(
            [pltpu.VMEM((nh, BL), jnp.int32),
             pltpu.VMEM((EMB, vocab), jnp.float32)]
            + [pltpu.VMEM((8, 8, 129), jnp.float32) for _ in range(2)]
            + [pltpu.SemaphoreType.DMA for _ in range(2)]
        ),
    )
    def gather_kernel(xt_hbm, tab_hbm, out_hbm, idx_v, tab_v,
                      tb0, tb1, os0, os1):
        tb = (tb0, tb1)
        osem = (os0, os1)

        wid = lax.axis_index("s") * NC + lax.axis_index("c")

        pltpu.sync_copy(xt_hbm.at[:, pl.ds(wid * BL, BL)], idx_v)
        pltpu.sync_copy(tab_hbm, tab_v)

        def start_write(h, s):
            pltpu.make_async_copy(
                tb[s].at[:, :, pl.ds(0, BL)], out_hbm.at[h, :, wid],
                osem[s]).start()

        def wait_write(s):
            pltpu.make_async_copy(
                tb[s].at[:, :, pl.ds(0, BL)], out_hbm.at[0, :, wid],
                osem[s]).wait()

        def lookup(h, s):
            for bc in range(BL // 16):
                idxv = idx_v[h, pl.ds(bc * 16, 16)]

                def e_body(j, _, idxv=idxv, bc=bc):
                    for u in range(8):
                        ev = jnp.full((16,), 0, jnp.int32) + (j * 8 + u)
                        v = plsc.load_gather(tab_v, [ev, idxv])
                        tb[s][j, u, pl.ds(bc * 16, 16)] = v
                    return 0

                lax.fori_loop(0, 8, e_body, 0)

        def group_body(hh, _):
            for s in range(2):
                h = hh * 2 + s

                @pl.when(hh >= 1)
                def _():
                    wait_write(s)

                lookup(h, s)
                start_write(h, s)

            return 0

        lax.fori_loop(0, nh // 2, group_body, 0)
        wait_write(0)
        wait_write(1)

    return gather_kernel


def kernel(x, embedding_weight):
    nb, nh = x.shape
    vocab = embedding_weight.shape[0]
    table_t = _relu_t_table(embedding_weight)
    xt = x.T
    y = _make_gather(nb, nh, vocab)(xt, table_t)
    return y.transpose(2, 4, 0, 1, 3).reshape(nb, nh, EMB)


# speedup vs baseline: 1.2339x; 1.2339x over previous
import functools

import jax
import jax.numpy as jnp
from jax import lax
from jax.experimental import pallas as pl
from jax.experimental.pallas import tpu as pltpu
from jax.experimental.pallas import tpu_sc as plsc

EMB = 64
NC = 2
NS = 16
NW = NC * NS
BL = 128


def _relu_t_body(w_ref, o_ref):
    o_ref[...] = jnp.maximum(w_ref[...], 0.0).T


def _relu_t_table(w):
    return pl.pallas_call(
        _relu_t_body,
        out_shape=jax.ShapeDtypeStruct((w.shape[1], w.shape[0]), w.dtype),
    )(w)


def _make_gather(nb, nh, vocab):
    nbt = nb // BL
    assert nbt == NW and nh % 2 == 0
    mesh = plsc.VectorSubcoreMesh(core_axis_name="c", subcore_axis_name="s")

    @functools.partial(
        pl.kernel,
        mesh=mesh,
        compiler_params=pltpu.CompilerParams(
            use_tc_tiling_on_sc=False, needs_layout_passes=False),
        out_type=jax.ShapeDtypeStruct((nh, 8, nbt, 8, BL), jnp.float32),
        scratch_types=(
            [pltpu.VMEM((nh, BL), jnp.int32),
             pltpu.VMEM((EMB * vocab,), jnp.float32)]
            + [pltpu.VMEM((8, 8, 129), jnp.float32) for _ in range(2)]
            + [pltpu.SemaphoreType.DMA for _ in range(2)]
        ),
    )
    def gather_kernel(xt_hbm, tab_hbm, out_hbm, idx_v, tab_v,
                      tb0, tb1, os0, os1):
        tb = (tb0, tb1)
        osem = (os0, os1)

        wid = lax.axis_index("s") * NC + lax.axis_index("c")

        pltpu.sync_copy(xt_hbm.at[:, pl.ds(wid * BL, BL)], idx_v)
        pltpu.sync_copy(tab_hbm, tab_v)

        def start_write(h, s):
            pltpu.make_async_copy(
                tb[s].at[:, :, pl.ds(0, BL)], out_hbm.at[h, :, wid],
                osem[s]).start()

        def wait_write(s):
            pltpu.make_async_copy(
                tb[s].at[:, :, pl.ds(0, BL)], out_hbm.at[0, :, wid],
                osem[s]).wait()

        def lookup(h, s):
            for bc in range(BL // 16):
                idxv = idx_v[h, pl.ds(bc * 16, 16)]

                def j_body(j, addrv, bc=bc):
                    for u in range(8):
                        v = plsc.load_gather(tab_v, [addrv])
                        tb[s][j, u, pl.ds(bc * 16, 16)] = v
                        addrv = addrv + vocab
                    return addrv

                lax.fori_loop(0, 8, j_body, idxv)

        def group_body(hh, _):
            for s in range(2):
                h = hh * 2 + s

                @pl.when(hh >= 1)
                def _():
                    wait_write(s)

                lookup(h, s)
                start_write(h, s)

            return 0

        lax.fori_loop(0, nh // 2, group_body, 0)
        wait_write(0)
        wait_write(1)

    return gather_kernel


def kernel(x, embedding_weight):
    nb, nh = x.shape
    vocab = embedding_weight.shape[0]
    table_t = _relu_t_table(embedding_weight)
    xt = x.T
    y = _make_gather(nb, nh, vocab)(xt, table_t.reshape(-1))
    return y.transpose(2, 4, 0, 1, 3).reshape(nb, nh, EMB)


# speedup vs baseline: 2.6246x; 2.1271x over previous
import functools

import jax
import jax.numpy as jnp
from jax import lax
from jax.experimental import pallas as pl
from jax.experimental.pallas import tpu as pltpu
from jax.experimental.pallas import tpu_sc as plsc

EMB = 64
NC = 2
NS = 16
NW = NC * NS
BL = 128


def _relu_t_body(w_ref, o_ref):
    o_ref[...] = jnp.maximum(w_ref[...], 0.0).T


def _relu_t_table(w):
    return pl.pallas_call(
        _relu_t_body,
        out_shape=jax.ShapeDtypeStruct((w.shape[1], w.shape[0]), w.dtype),
    )(w)


def _make_gather(nb, nh, vocab):
    nbt = nb // BL
    assert nbt == NW and nh % 2 == 0
    mesh = plsc.VectorSubcoreMesh(core_axis_name="c", subcore_axis_name="s")

    @functools.partial(
        pl.kernel,
        mesh=mesh,
        compiler_params=pltpu.CompilerParams(
            use_tc_tiling_on_sc=False, needs_layout_passes=False),
        out_type=jax.ShapeDtypeStruct((nh, 8, nbt, 8, BL), jnp.float32),
        scratch_types=(
            [pltpu.VMEM((nh, BL), jnp.int32),
             pltpu.VMEM((EMB * vocab,), jnp.float32)]
            + [pltpu.VMEM((8, 8, 129), jnp.float32) for _ in range(2)]
            + [pltpu.SemaphoreType.DMA for _ in range(2)]
        ),
    )
    def gather_kernel(xt_hbm, tab_hbm, out_hbm, idx_v, tab_v,
                      tb0, tb1, os0, os1):
        tb = (tb0, tb1)
        osem = (os0, os1)

        wid = lax.axis_index("s") * NC + lax.axis_index("c")

        pltpu.sync_copy(xt_hbm.at[:, pl.ds(wid * BL, BL)], idx_v)
        pltpu.sync_copy(tab_hbm, tab_v)

        def start_write(h, s):
            pltpu.make_async_copy(
                tb[s].at[:, :, pl.ds(0, BL)], out_hbm.at[h, :, wid],
                osem[s]).start()

        def wait_write(s):
            pltpu.make_async_copy(
                tb[s].at[:, :, pl.ds(0, BL)], out_hbm.at[0, :, wid],
                osem[s]).wait()

        def lookup(h, s):
            nch = 4
            for bcg in range(BL // 16 // nch):
                idxvs = tuple(
                    idx_v[h, pl.ds((bcg * nch + c) * 16, 16)]
                    for c in range(nch))

                def j_body(j, addrs, bcg=bcg):
                    for u in range(8):
                        vs = [plsc.load_gather(tab_v, [a]) for a in addrs]
                        for c, v in enumerate(vs):
                            tb[s][j, u, pl.ds((bcg * nch + c) * 16, 16)] = v
                        addrs = tuple(a + vocab for a in addrs)
                    return addrs

                lax.fori_loop(0, 8, j_body, idxvs)

        def group_body(hh, _):
            for s in range(2):
                h = hh * 2 + s

                @pl.when(hh >= 1)
                def _():
                    wait_write(s)

                lookup(h, s)
                start_write(h, s)

            return 0

        lax.fori_loop(0, nh // 2, group_body, 0)
        wait_write(0)
        wait_write(1)

    return gather_kernel


def kernel(x, embedding_weight):
    nb, nh = x.shape
    vocab = embedding_weight.shape[0]
    table_t = _relu_t_table(embedding_weight)
    xt = x.T
    y = _make_gather(nb, nh, vocab)(xt, table_t.reshape(-1))
    return y.transpose(2, 4, 0, 1, 3).reshape(nb, nh, EMB)


# speedup vs baseline: 3.2274x; 1.2297x over previous
import functools

import jax
import jax.numpy as jnp
from jax import lax
from jax.experimental import pallas as pl
from jax.experimental.pallas import tpu as pltpu
from jax.experimental.pallas import tpu_sc as plsc

EMB = 64
NC = 2
NS = 16
NW = NC * NS
BL = 128


def _relu_t_body(w_ref, o_ref):
    o_ref[...] = jnp.maximum(w_ref[...], 0.0).T


def _relu_t_table(w):
    return pl.pallas_call(
        _relu_t_body,
        out_shape=jax.ShapeDtypeStruct((w.shape[1], w.shape[0]), w.dtype),
    )(w)


def _make_gather(nb, nh, vocab):
    nbt = nb // BL
    assert nbt == NW and nh % 2 == 0
    mesh = plsc.VectorSubcoreMesh(core_axis_name="c", subcore_axis_name="s")

    @functools.partial(
        pl.kernel,
        mesh=mesh,
        compiler_params=pltpu.CompilerParams(
            use_tc_tiling_on_sc=False, needs_layout_passes=False),
        out_type=jax.ShapeDtypeStruct((nh, 8, nbt, 8, BL), jnp.float32),
        scratch_types=(
            [pltpu.VMEM((nh, BL), jnp.int32),
             pltpu.VMEM((EMB * vocab,), jnp.float32)]
            + [pltpu.VMEM((8, 8, 129), jnp.float32) for _ in range(2)]
            + [pltpu.SemaphoreType.DMA for _ in range(2)]
        ),
    )
    def gather_kernel(xt_hbm, tab_hbm, out_hbm, idx_v, tab_v,
                      tb0, tb1, os0, os1):
        tb = (tb0, tb1)
        osem = (os0, os1)

        wid = lax.axis_index("s") * NC + lax.axis_index("c")

        pltpu.sync_copy(xt_hbm.at[:, pl.ds(wid * BL, BL)], idx_v)
        pltpu.sync_copy(tab_hbm, tab_v)

        def start_write(h, s):
            pltpu.make_async_copy(
                tb[s].at[:, :, pl.ds(0, BL)], out_hbm.at[h, :, wid],
                osem[s]).start()

        def wait_write(s):
            pltpu.make_async_copy(
                tb[s].at[:, :, pl.ds(0, BL)], out_hbm.at[0, :, wid],
                osem[s]).wait()

        def lookup(h, s):
            nch = 8
            for bcg in range(BL // 16 // nch):
                idxvs = tuple(
                    idx_v[h, pl.ds((bcg * nch + c) * 16, 16)]
                    for c in range(nch))

                def j_body(j, addrs, bcg=bcg):
                    for u in range(8):
                        vs = [plsc.load_gather(tab_v, [a]) for a in addrs]
                        for c, v in enumerate(vs):
                            tb[s][j, u, pl.ds((bcg * nch + c) * 16, 16)] = v
                        addrs = tuple(a + vocab for a in addrs)
                    return addrs

                lax.fori_loop(0, 8, j_body, idxvs)

        def group_body(hh, _):
            for s in range(2):
                h = hh * 2 + s

                @pl.when(hh >= 1)
                def _():
                    wait_write(s)

                lookup(h, s)
                start_write(h, s)

            return 0

        lax.fori_loop(0, nh // 2, group_body, 0)
        wait_write(0)
        wait_write(1)

    return gather_kernel


def kernel(x, embedding_weight):
    nb, nh = x.shape
    vocab = embedding_weight.shape[0]
    table_t = _relu_t_table(embedding_weight)
    xt = x.T
    y = _make_gather(nb, nh, vocab)(xt, table_t.reshape(-1))
    return y.transpose(2, 4, 0, 1, 3).reshape(nb, nh, EMB)


# speedup vs baseline: 5.9299x; 1.8373x over previous
import functools

import jax
import jax.numpy as jnp
from jax import lax
from jax.experimental import pallas as pl
from jax.experimental.pallas import tpu as pltpu
from jax.experimental.pallas import tpu_sc as plsc

EMB = 64
NC = 2
NS = 16
NW = NC * NS
BL = 128


def _relu_t_body(w_ref, o_ref):
    o_ref[...] = jnp.maximum(w_ref[...], 0.0).T


def _relu_t_table(w):
    return pl.pallas_call(
        _relu_t_body,
        out_shape=jax.ShapeDtypeStruct((w.shape[1], w.shape[0]), w.dtype),
    )(w)


def _make_gather(nb, nh, vocab):
    nbt = nb // BL
    assert nbt == NW and nh % 2 == 0
    mesh = plsc.VectorSubcoreMesh(core_axis_name="c", subcore_axis_name="s")

    @functools.partial(
        pl.kernel,
        mesh=mesh,
        compiler_params=pltpu.CompilerParams(
            use_tc_tiling_on_sc=False, needs_layout_passes=False),
        out_type=jax.ShapeDtypeStruct((nh, 8, nbt, 8, BL), jnp.float32),
        scratch_types=(
            [pltpu.VMEM((nh, BL), jnp.int32),
             pltpu.VMEM((EMB * vocab,), jnp.float32)]
            + [pltpu.VMEM((8, 8, 129), jnp.float32) for _ in range(2)]
            + [pltpu.SemaphoreType.DMA for _ in range(2)]
        ),
    )
    def gather_kernel(xt_hbm, tab_hbm, out_hbm, idx_v, tab_v,
                      tb0, tb1, os0, os1):
        tb = (tb0, tb1)
        osem = (os0, os1)

        wid = lax.axis_index("s") * NC + lax.axis_index("c")

        pltpu.sync_copy(xt_hbm.at[:, pl.ds(wid * BL, BL)], idx_v)
        pltpu.sync_copy(tab_hbm, tab_v)

        def start_write(h, s):
            pltpu.make_async_copy(
                tb[s].at[:, :, pl.ds(0, BL)], out_hbm.at[h, :, wid],
                osem[s]).start()

        def wait_write(s):
            pltpu.make_async_copy(
                tb[s].at[:, :, pl.ds(0, BL)], out_hbm.at[0, :, wid],
                osem[s]).wait()

        def lookup(h, s):
            nch = 16
            for bcg in range(BL // 16 // nch):
                idxvs = tuple(
                    idx_v[h, pl.ds((bcg * nch + c) * 16, 16)]
                    for c in range(nch))

                def j_body(j, addrs, bcg=bcg):
                    for u in range(8):
                        vs = [plsc.load_gather(tab_v, [a]) for a in addrs]
                        for c, v in enumerate(vs):
                            tb[s][j, u, pl.ds((bcg * nch + c) * 16, 16)] = v
                        addrs = tuple(a + vocab for a in addrs)
                    return addrs

                lax.fori_loop(0, 8, j_body, idxvs)

        def group_body(hh, _):
            for s in range(2):
                h = hh * 2 + s

                @pl.when(hh >= 1)
                def _():
                    wait_write(s)

                lookup(h, s)
                start_write(h, s)

            return 0

        lax.fori_loop(0, nh // 2, group_body, 0)
        wait_write(0)
        wait_write(1)

    return gather_kernel


def kernel(x, embedding_weight):
    nb, nh = x.shape
    vocab = embedding_weight.shape[0]
    table_t = _relu_t_table(embedding_weight)
    xt = x.T
    y = _make_gather(nb, nh, vocab)(xt, table_t.reshape(-1))
    return y.transpose(2, 4, 0, 1, 3).reshape(nb, nh, EMB)
